# Initial kernel scaffold; baseline (speedup 1.0000x reference)
import functools

import jax
import jax.numpy as jnp
from jax import lax
from jax.experimental import pallas as pl
from jax.experimental.pallas import tpu as pltpu
from jax.experimental.pallas import tpu_sc as plsc

N = 10000
E = 320000
D_IN = 128
D_HID = 16
D_OUT = 7

NC = 2
NS = 16
NW = NC * NS
EB = 128
NBATCH = E // EB
BPT = -(-NBATCH // NW)
RPS = N // NS


def _sc_mesh():
    return plsc.VectorSubcoreMesh(
        core_axis_name="c", subcore_axis_name="s",
        num_cores=NC, num_subcores=NS)


def _deg_scatter(dst, ones_rows, zeros_rows):
    @functools.partial(
        pl.kernel,
        out_type=jax.ShapeDtypeStruct((NC * N, 16), jnp.float32),
        mesh=_sc_mesh(),
        scratch_types=[
            pltpu.VMEM((1, EB), jnp.int32),
            pltpu.VMEM((EB, 16), jnp.float32),
            pltpu.VMEM_SHARED((N, 16), jnp.float32),
        ],
    )
    def k(dst_h, ones_h, zeros_h, out_h, didx, ones_v, acc):
        cid = lax.axis_index("c")
        sid = lax.axis_index("s")
        wid = sid * NC + cid
        pltpu.sync_copy(ones_h, ones_v)
        pltpu.sync_copy(zeros_h, acc.at[pl.ds(sid * RPS, RPS)])
        plsc.subcore_barrier()

        def body(t, carry):
            b = wid + t * NW

            @pl.when(b < NBATCH)
            def _():
                pltpu.sync_copy(dst_h.at[pl.ds(b * EB, EB)], didx.at[0])
                pltpu.sync_copy(ones_v, acc.at[didx.at[0]], add=True)

            return carry

        lax.fori_loop(0, BPT, body, 0)
        plsc.subcore_barrier()
        pltpu.sync_copy(acc.at[pl.ds(sid * RPS, RPS)],
                        out_h.at[pl.ds(cid * N + sid * RPS, RPS)])

    return k(dst, ones_rows, zeros_rows)


def _edge_scatter(table, src, dst, zeros_rows):
    @functools.partial(
        pl.kernel,
        out_type=jax.ShapeDtypeStruct((NC * N, 16), jnp.float32),
        mesh=_sc_mesh(),
        scratch_types=[
            pltpu.VMEM((1, EB), jnp.int32),
            pltpu.VMEM((1, EB), jnp.int32),
            pltpu.VMEM((EB, 16), jnp.float32),
            pltpu.VMEM_SHARED((N, 16), jnp.float32),
            pltpu.SemaphoreType.DMA,
        ],
    )
    def k(table_h, src_h, dst_h, zeros_h, out_h, sidx, didx, rows, acc, sem):
        cid = lax.axis_index("c")
        sid = lax.axis_index("s")
        wid = sid * NC + cid
        pltpu.sync_copy(zeros_h, acc.at[pl.ds(sid * RPS, RPS)])
        plsc.subcore_barrier()

        def body(t, carry):
            b = wid + t * NW

            @pl.when(b < NBATCH)
            def _():
                pltpu.sync_copy(src_h.at[pl.ds(b * EB, EB)], sidx.at[0])
                pltpu.sync_copy(dst_h.at[pl.ds(b * EB, EB)], didx.at[0])
                pltpu.async_copy(table_h.at[sidx.at[0]], rows, sem).wait()
                pltpu.sync_copy(rows, acc.at[didx.at[0]], add=True)

            return carry

        lax.fori_loop(0, BPT, body, 0)
        plsc.subcore_barrier()
        pltpu.sync_copy(acc.at[pl.ds(sid * RPS, RPS)],
                        out_h.at[pl.ds(cid * N + sid * RPS, RPS)])

    return k(table, src, dst, zeros_rows)


def _tc_prep1(degp, x, w1):
    def body(degp_ref, x_ref, w_ref, g_ref, dinv_ref):
        deg = degp_ref[0] + degp_ref[1] + 1.0
        dinv = lax.rsqrt(deg)
        h = jnp.dot(x_ref[...], w_ref[...],
                    preferred_element_type=jnp.float32)
        g_ref[...] = h * dinv
        dinv_ref[...] = dinv

    return pl.pallas_call(
        body,
        out_shape=(jax.ShapeDtypeStruct((N, 16), jnp.float32),
                   jax.ShapeDtypeStruct((N, 16), jnp.float32)),
    )(degp, x, w1)


def _tc_mid(accp, g1, dinv, w2p, b1r):
    def body(accp_ref, g_ref, dinv_ref, w_ref, b_ref, g2_ref):
        z = dinv_ref[...] * (accp_ref[0] + accp_ref[1] + g_ref[...])
        z = z + b_ref[...]
        h = jnp.maximum(z, 0.0)
        h2 = jnp.dot(h, w_ref[...], preferred_element_type=jnp.float32)
        g2_ref[...] = h2 * dinv_ref[...]

    return pl.pallas_call(
        body,
        out_shape=jax.ShapeDtypeStruct((N, 16), jnp.float32),
    )(accp, g1, dinv, w2p, b1r)


def _tc_final(accp, g2, dinv, b2r):
    def body(accp_ref, g_ref, dinv_ref, b_ref, o_ref):
        z = dinv_ref[...] * (accp_ref[0] + accp_ref[1] + g_ref[...])
        z = z + b_ref[...]
        col = lax.broadcasted_iota(jnp.int32, (N, 16), 1)
        zm = jnp.where(col < D_OUT, z, -jnp.inf)
        m = jnp.max(zm, axis=1, keepdims=True)
        e = jnp.exp(zm - m)
        lse = jnp.log(jnp.sum(e, axis=1, keepdims=True)) + m
        o_ref[...] = (z - lse)[:, :D_OUT]

    return pl.pallas_call(
        body,
        out_shape=jax.ShapeDtypeStruct((N, D_OUT), jnp.float32),
    )(accp, g2, dinv, b2r)


@jax.jit
def kernel(x, edge_index, W1, b1, W2, b2):
    src = edge_index[0].astype(jnp.int32)
    dst = edge_index[1].astype(jnp.int32)
    zeros_rows = jnp.zeros((RPS, 16), jnp.float32)
    ones_rows = jnp.ones((EB, 16), jnp.float32)

    degp = _deg_scatter(dst, ones_rows, zeros_rows).reshape(NC, N, 16)
    g1, dinv = _tc_prep1(degp, x, W1)

    acc1 = _edge_scatter(g1, src, dst, zeros_rows).reshape(NC, N, 16)

    w2p = jnp.pad(W2, ((0, 0), (0, 16 - D_OUT)))
    b1r = b1.reshape(1, 16)
    b2r = jnp.pad(b2, (0, 16 - D_OUT)).reshape(1, 16)

    g2 = _tc_mid(acc1, g1, dinv, w2p, b1r)
    acc2 = _edge_scatter(g2, src, dst, zeros_rows).reshape(NC, N, 16)
    return _tc_final(acc2, g2, dinv, b2r)


# speedup vs baseline: 20.3035x; 20.3035x over previous
import functools

import jax
import jax.numpy as jnp
from jax import lax
from jax.experimental import pallas as pl
from jax.experimental.pallas import tpu as pltpu
from jax.experimental.pallas import tpu_sc as plsc

N = 10000
E = 320000
D_IN = 128
D_HID = 16
D_OUT = 7

NC = 2
NS = 16
NW = NC * NS
EB = 128
NBATCH = E // EB
BPT = -(-NBATCH // NW)
NP = 10240
RPS = NP // NS


def _sc_mesh():
    return plsc.VectorSubcoreMesh(
        core_axis_name="c", subcore_axis_name="s",
        num_cores=NC, num_subcores=NS)


def _deg_scatter(dst, ones_rows, zeros_rows):
    @functools.partial(
        pl.kernel,
        out_type=jax.ShapeDtypeStruct((NC * NP, 16), jnp.float32),
        mesh=_sc_mesh(),
        scratch_types=[
            pltpu.VMEM((1, EB), jnp.int32),
            pltpu.VMEM((EB, 16), jnp.float32),
            pltpu.VMEM_SHARED((NP, 16), jnp.float32),
        ],
        compiler_params=pltpu.CompilerParams(use_tc_tiling_on_sc=False),
    )
    def k(dst_h, ones_h, zeros_h, out_h, didx, ones_v, acc):
        cid = lax.axis_index("c")
        sid = lax.axis_index("s")
        wid = sid * NC + cid
        pltpu.sync_copy(ones_h, ones_v)
        pltpu.sync_copy(zeros_h, acc.at[pl.ds(sid * RPS, RPS)])
        plsc.subcore_barrier()

        def body(t, carry):
            b = wid + t * NW

            @pl.when(b < NBATCH)
            def _():
                pltpu.sync_copy(dst_h.at[pl.ds(b * EB, EB)], didx.at[0])
                pltpu.sync_copy(ones_v, acc.at[didx.at[0]], add=True)

            return carry

        lax.fori_loop(0, BPT, body, 0)
        plsc.subcore_barrier()
        pltpu.sync_copy(acc.at[pl.ds(sid * RPS, RPS)],
                        out_h.at[pl.ds(cid * NP + sid * RPS, RPS)])

    return k(dst, ones_rows, zeros_rows)


def _edge_scatter(table, src, dst, zeros_rows):
    @functools.partial(
        pl.kernel,
        out_type=jax.ShapeDtypeStruct((NC * NP, 16), jnp.float32),
        mesh=_sc_mesh(),
        scratch_types=[
            pltpu.VMEM((1, EB), jnp.int32),
            pltpu.VMEM((1, EB), jnp.int32),
            pltpu.VMEM((EB, 16), jnp.float32),
            pltpu.VMEM_SHARED((NP, 16), jnp.float32),
            pltpu.SemaphoreType.DMA,
        ],
        compiler_params=pltpu.CompilerParams(use_tc_tiling_on_sc=False),
    )
    def k(table_h, src_h, dst_h, zeros_h, out_h, sidx, didx, rows, acc, sem):
        cid = lax.axis_index("c")
        sid = lax.axis_index("s")
        wid = sid * NC + cid
        pltpu.sync_copy(zeros_h, acc.at[pl.ds(sid * RPS, RPS)])
        plsc.subcore_barrier()

        def body(t, carry):
            b = wid + t * NW

            @pl.when(b < NBATCH)
            def _():
                pltpu.sync_copy(src_h.at[pl.ds(b * EB, EB)], sidx.at[0])
                pltpu.sync_copy(dst_h.at[pl.ds(b * EB, EB)], didx.at[0])
                pltpu.async_copy(table_h.at[sidx.at[0]], rows, sem).wait()
                pltpu.sync_copy(rows, acc.at[didx.at[0]], add=True)

            return carry

        lax.fori_loop(0, BPT, body, 0)
        plsc.subcore_barrier()
        pltpu.sync_copy(acc.at[pl.ds(sid * RPS, RPS)],
                        out_h.at[pl.ds(cid * NP + sid * RPS, RPS)])

    return k(table, src, dst, zeros_rows)


def _tc_prep1(degp, x, w1):
    def body(degp_ref, x_ref, w_ref, g_ref, dinv_ref):
        deg = degp_ref[0] + degp_ref[1] + 1.0
        dinv = lax.rsqrt(deg)
        h = jnp.dot(x_ref[...], w_ref[...],
                    preferred_element_type=jnp.float32)
        g_ref[...] = h * dinv
        dinv_ref[...] = dinv

    return pl.pallas_call(
        body,
        out_shape=(jax.ShapeDtypeStruct((N, 16), jnp.float32),
                   jax.ShapeDtypeStruct((N, 16), jnp.float32)),
    )(degp, x, w1)


def _tc_mid(accp, g1, dinv, w2p, b1r):
    def body(accp_ref, g_ref, dinv_ref, w_ref, b_ref, g2_ref):
        z = dinv_ref[...] * (accp_ref[0] + accp_ref[1] + g_ref[...])
        z = z + b_ref[...]
        h = jnp.maximum(z, 0.0)
        h2 = jnp.dot(h, w_ref[...], preferred_element_type=jnp.float32)
        g2_ref[...] = h2 * dinv_ref[...]

    return pl.pallas_call(
        body,
        out_shape=jax.ShapeDtypeStruct((N, 16), jnp.float32),
    )(accp, g1, dinv, w2p, b1r)


def _tc_final(accp, g2, dinv, b2r):
    def body(accp_ref, g_ref, dinv_ref, b_ref, o_ref):
        z = dinv_ref[...] * (accp_ref[0] + accp_ref[1] + g_ref[...])
        z = z + b_ref[...]
        col = lax.broadcasted_iota(jnp.int32, (N, 16), 1)
        zm = jnp.where(col < D_OUT, z, -jnp.inf)
        m = jnp.max(zm, axis=1, keepdims=True)
        e = jnp.exp(zm - m)
        lse = jnp.log(jnp.sum(e, axis=1, keepdims=True)) + m
        o_ref[...] = (z - lse)[:, :D_OUT]

    return pl.pallas_call(
        body,
        out_shape=jax.ShapeDtypeStruct((N, D_OUT), jnp.float32),
    )(accp, g2, dinv, b2r)


@jax.jit
def kernel(x, edge_index, W1, b1, W2, b2):
    src = edge_index[0].astype(jnp.int32)
    dst = edge_index[1].astype(jnp.int32)
    zeros_rows = jnp.zeros((RPS, 16), jnp.float32)
    ones_rows = jnp.ones((EB, 16), jnp.float32)

    degp = _deg_scatter(dst, ones_rows, zeros_rows).reshape(NC, NP, 16)[:, :N]
    g1, dinv = _tc_prep1(degp, x, W1)

    acc1 = _edge_scatter(g1, src, dst, zeros_rows).reshape(NC, NP, 16)[:, :N]

    w2p = jnp.pad(W2, ((0, 0), (0, 16 - D_OUT)))
    b1r = b1.reshape(1, 16)
    b2r = jnp.pad(b2, (0, 16 - D_OUT)).reshape(1, 16)

    g2 = _tc_mid(acc1, g1, dinv, w2p, b1r)
    acc2 = _edge_scatter(g2, src, dst, zeros_rows).reshape(NC, NP, 16)[:, :N]
    return _tc_final(acc2, g2, dinv, b2r)


# speedup vs baseline: 31.2955x; 1.5414x over previous
import functools

import jax
import jax.numpy as jnp
from jax import lax
from jax.experimental import pallas as pl
from jax.experimental.pallas import tpu as pltpu
from jax.experimental.pallas import tpu_sc as plsc

N = 10000
E = 320000
D_IN = 128
D_HID = 16
D_OUT = 7

NC = 2
NS = 16
NW = NC * NS
EB = 128
NBATCH = E // EB
BPT = -(-NBATCH // NW)
NP = 10240
RPS = NP // NS


def _sc_mesh():
    return plsc.VectorSubcoreMesh(
        core_axis_name="c", subcore_axis_name="s",
        num_cores=NC, num_subcores=NS)


def _deg_scatter(dst, ones_rows, zeros_rows):
    @functools.partial(
        pl.kernel,
        out_type=jax.ShapeDtypeStruct((NC * NP, 16), jnp.float32),
        mesh=_sc_mesh(),
        scratch_types=[
            pltpu.VMEM((2, EB), jnp.int32),
            pltpu.VMEM((EB, 16), jnp.float32),
            pltpu.VMEM_SHARED((NP, 16), jnp.float32),
            pltpu.SemaphoreType.DMA((2,)),
        ],
        compiler_params=pltpu.CompilerParams(use_tc_tiling_on_sc=False),
    )
    def k(dst_h, ones_h, zeros_h, out_h, didx, ones_v, acc, sem_i):
        cid = lax.axis_index("c")
        sid = lax.axis_index("s")
        wid = sid * NC + cid
        pltpu.sync_copy(ones_h, ones_v)
        pltpu.sync_copy(zeros_h, acc.at[pl.ds(sid * RPS, RPS)])
        plsc.subcore_barrier()

        def valid(t):
            return (wid + t * NW) < NBATCH

        def boff(t):
            return (wid + t * NW) * EB

        def idx_copy(t, s):
            return pltpu.make_async_copy(
                dst_h.at[pl.ds(boff(t), EB)], didx.at[s], sem_i.at[s])

        def issue_idx(t, s):
            @pl.when(valid(t))
            def _():
                idx_copy(t, s).start()

        def scatter(t, s):
            @pl.when(valid(t))
            def _():
                idx_copy(t, s).wait()
                pltpu.sync_copy(ones_v, acc.at[didx.at[s]], add=True)

        issue_idx(0, 0)
        issue_idx(1, 1)

        def body(T2, carry):
            T = T2 * 2
            for s in range(2):
                t = T + s
                scatter(t, s)
                issue_idx(t + 2, s)
            return carry

        lax.fori_loop(0, BPT // 2 + 1, body, 0)
        plsc.subcore_barrier()
        pltpu.sync_copy(acc.at[pl.ds(sid * RPS, RPS)],
                        out_h.at[pl.ds(cid * NP + sid * RPS, RPS)])

    return k(dst, ones_rows, zeros_rows)


def _edge_scatter(table, src, dst, zeros_rows):
    @functools.partial(
        pl.kernel,
        out_type=jax.ShapeDtypeStruct((NC * NP, 16), jnp.float32),
        mesh=_sc_mesh(),
        scratch_types=[
            pltpu.VMEM((3, EB), jnp.int32),
            pltpu.VMEM((3, EB), jnp.int32),
            pltpu.VMEM((3, EB, 16), jnp.float32),
            pltpu.VMEM_SHARED((NP, 16), jnp.float32),
            pltpu.SemaphoreType.DMA((3,)),
            pltpu.SemaphoreType.DMA((3,)),
        ],
        compiler_params=pltpu.CompilerParams(use_tc_tiling_on_sc=False),
    )
    def k(table_h, src_h, dst_h, zeros_h, out_h,
          sidx, didx, rows, acc, sem_i, sem_g):
        cid = lax.axis_index("c")
        sid = lax.axis_index("s")
        wid = sid * NC + cid
        pltpu.sync_copy(zeros_h, acc.at[pl.ds(sid * RPS, RPS)])
        plsc.subcore_barrier()

        def valid(t):
            return jnp.logical_and(t >= 0, (wid + t * NW) < NBATCH)

        def boff(t):
            return (wid + t * NW) * EB

        def sidx_copy(t, s):
            return pltpu.make_async_copy(
                src_h.at[pl.ds(boff(t), EB)], sidx.at[s], sem_i.at[s])

        def didx_copy(t, s):
            return pltpu.make_async_copy(
                dst_h.at[pl.ds(boff(t), EB)], didx.at[s], sem_i.at[s])

        def gather_copy(s):
            return pltpu.make_async_copy(
                table_h.at[sidx.at[s]], rows.at[s], sem_g.at[s])

        def issue_idx(t, s):
            @pl.when(valid(t))
            def _():
                sidx_copy(t, s).start()
                didx_copy(t, s).start()

        def start_gather(t, s):
            @pl.when(valid(t))
            def _():
                sidx_copy(t, s).wait()
                didx_copy(t, s).wait()
                gather_copy(s).start()

        def scatter(t, s):
            @pl.when(valid(t))
            def _():
                gather_copy(s).wait()
                pltpu.sync_copy(rows.at[s], acc.at[didx.at[s]], add=True)

        issue_idx(0, 0)
        issue_idx(1, 1)

        def body(T3, carry):
            T = T3 * 3
            for s in range(3):
                t = T + s
                scatter(t - 1, (s + 2) % 3)
                issue_idx(t + 2, (s + 2) % 3)
                start_gather(t, s)
            return carry

        lax.fori_loop(0, BPT // 3 + 1, body, 0)
        plsc.subcore_barrier()
        pltpu.sync_copy(acc.at[pl.ds(sid * RPS, RPS)],
                        out_h.at[pl.ds(cid * NP + sid * RPS, RPS)])

    return k(table, src, dst, zeros_rows)


def _tc_prep1(degp, x, w1):
    def body(degp_ref, x_ref, w_ref, g_ref, dinv_ref):
        deg = degp_ref[0] + degp_ref[1] + 1.0
        dinv = lax.rsqrt(deg)
        h = jnp.dot(x_ref[...], w_ref[...],
                    preferred_element_type=jnp.float32)
        g_ref[...] = h * dinv
        dinv_ref[...] = dinv

    return pl.pallas_call(
        body,
        out_shape=(jax.ShapeDtypeStruct((N, 16), jnp.float32),
                   jax.ShapeDtypeStruct((N, 16), jnp.float32)),
    )(degp, x, w1)


def _tc_mid(accp, g1, dinv, w2p, b1r):
    def body(accp_ref, g_ref, dinv_ref, w_ref, b_ref, g2_ref):
        z = dinv_ref[...] * (accp_ref[0] + accp_ref[1] + g_ref[...])
        z = z + b_ref[...]
        h = jnp.maximum(z, 0.0)
        h2 = jnp.dot(h, w_ref[...], preferred_element_type=jnp.float32)
        g2_ref[...] = h2 * dinv_ref[...]

    return pl.pallas_call(
        body,
        out_shape=jax.ShapeDtypeStruct((N, 16), jnp.float32),
    )(accp, g1, dinv, w2p, b1r)


def _tc_final(accp, g2, dinv, b2r):
    def body(accp_ref, g_ref, dinv_ref, b_ref, o_ref):
        z = dinv_ref[...] * (accp_ref[0] + accp_ref[1] + g_ref[...])
        z = z + b_ref[...]
        col = lax.broadcasted_iota(jnp.int32, (N, 16), 1)
        zm = jnp.where(col < D_OUT, z, -jnp.inf)
        m = jnp.max(zm, axis=1, keepdims=True)
        e = jnp.exp(zm - m)
        lse = jnp.log(jnp.sum(e, axis=1, keepdims=True)) + m
        o_ref[...] = (z - lse)[:, :D_OUT]

    return pl.pallas_call(
        body,
        out_shape=jax.ShapeDtypeStruct((N, D_OUT), jnp.float32),
    )(accp, g2, dinv, b2r)


@jax.jit
def kernel(x, edge_index, W1, b1, W2, b2):
    src = edge_index[0].astype(jnp.int32)
    dst = edge_index[1].astype(jnp.int32)
    zeros_rows = jnp.zeros((RPS, 16), jnp.float32)
    ones_rows = jnp.ones((EB, 16), jnp.float32)

    degp = _deg_scatter(dst, ones_rows, zeros_rows).reshape(NC, NP, 16)[:, :N]
    g1, dinv = _tc_prep1(degp, x, W1)

    acc1 = _edge_scatter(g1, src, dst, zeros_rows).reshape(NC, NP, 16)[:, :N]

    w2p = jnp.pad(W2, ((0, 0), (0, 16 - D_OUT)))
    b1r = b1.reshape(1, 16)
    b2r = jnp.pad(b2, (0, 16 - D_OUT)).reshape(1, 16)

    g2 = _tc_mid(acc1, g1, dinv, w2p, b1r)
    acc2 = _edge_scatter(g2, src, dst, zeros_rows).reshape(NC, NP, 16)[:, :N]
    return _tc_final(acc2, g2, dinv, b2r)


# speedup vs baseline: 41.9736x; 1.3412x over previous
import functools

import jax
import jax.numpy as jnp
from jax import lax
from jax.experimental import pallas as pl
from jax.experimental.pallas import tpu as pltpu
from jax.experimental.pallas import tpu_sc as plsc

N = 10000
E = 320000
D_IN = 128
D_HID = 16
D_OUT = 7

NC = 2
NS = 16
NW = NC * NS
EB = 128
NBATCH = E // EB
BPT = -(-NBATCH // NW)
NP = 10240
RPS = NP // NS


def _sc_mesh():
    return plsc.VectorSubcoreMesh(
        core_axis_name="c", subcore_axis_name="s",
        num_cores=NC, num_subcores=NS)


def _deg_scatter(dst, ones_rows, zeros_rows):
    @functools.partial(
        pl.kernel,
        out_type=jax.ShapeDtypeStruct((NC * NP, 16), jnp.float32),
        mesh=_sc_mesh(),
        scratch_types=[
            pltpu.VMEM((4, EB), jnp.int32),
            pltpu.VMEM((EB, 16), jnp.float32),
            pltpu.VMEM_SHARED((NP, 16), jnp.float32),
            pltpu.SemaphoreType.DMA((4,)),
            pltpu.SemaphoreType.DMA((4,)),
        ],
        compiler_params=pltpu.CompilerParams(use_tc_tiling_on_sc=False),
    )
    def k(dst_h, ones_h, zeros_h, out_h, didx, ones_v, acc, sem_i, sem_sc):
        cid = lax.axis_index("c")
        sid = lax.axis_index("s")
        wid = sid * NC + cid
        pltpu.sync_copy(ones_h, ones_v)
        pltpu.sync_copy(zeros_h, acc.at[pl.ds(sid * RPS, RPS)])
        plsc.subcore_barrier()

        def valid(t):
            return jnp.logical_and(t >= 0, (wid + t * NW) < NBATCH)

        def boff(t):
            return (wid + t * NW) * EB

        def idx_copy(t, s):
            return pltpu.make_async_copy(
                dst_h.at[pl.ds(boff(t), EB)], didx.at[s], sem_i.at[s])

        def sc_desc(s):
            return pltpu.make_async_copy(
                ones_v, acc.at[didx.at[s]], sem_sc.at[s])

        def issue_idx(t, s):
            @pl.when(valid(t))
            def _():
                idx_copy(t, s).start()

        def wait_sc(t, s):
            @pl.when(valid(t))
            def _():
                sc_desc(s).wait()

        def scatter(t, s):
            @pl.when(valid(t))
            def _():
                idx_copy(t, s).wait()
                pltpu.async_copy(ones_v, acc.at[didx.at[s]], sem_sc.at[s],
                                 add=True)

        issue_idx(0, 0)
        issue_idx(1, 1)

        def body(T4, carry):
            T = T4 * 4
            for s in range(4):
                t = T + s
                wait_sc(t - 2, (s + 2) % 4)
                issue_idx(t + 2, (s + 2) % 4)
                scatter(t, s)
            return carry

        lax.fori_loop(0, BPT // 4 + 1, body, 0)
        wait_sc(BPT - 1, (BPT - 1) % 4)
        wait_sc(BPT, BPT % 4)
        plsc.subcore_barrier()
        pltpu.sync_copy(acc.at[pl.ds(sid * RPS, RPS)],
                        out_h.at[pl.ds(cid * NP + sid * RPS, RPS)])

    return k(dst, ones_rows, zeros_rows)


def _edge_scatter(table, src, dst, zeros_rows):
    @functools.partial(
        pl.kernel,
        out_type=jax.ShapeDtypeStruct((NC * NP, 16), jnp.float32),
        mesh=_sc_mesh(),
        scratch_types=[
            pltpu.VMEM((4, EB), jnp.int32),
            pltpu.VMEM((4, EB), jnp.int32),
            pltpu.VMEM((4, EB, 16), jnp.float32),
            pltpu.VMEM_SHARED((NP, 16), jnp.float32),
            pltpu.SemaphoreType.DMA((4,)),
            pltpu.SemaphoreType.DMA((4,)),
            pltpu.SemaphoreType.DMA((4,)),
        ],
        compiler_params=pltpu.CompilerParams(use_tc_tiling_on_sc=False),
    )
    def k(table_h, src_h, dst_h, zeros_h, out_h,
          sidx, didx, rows, acc, sem_i, sem_g, sem_sc):
        cid = lax.axis_index("c")
        sid = lax.axis_index("s")
        wid = sid * NC + cid
        pltpu.sync_copy(zeros_h, acc.at[pl.ds(sid * RPS, RPS)])
        plsc.subcore_barrier()

        def valid(t):
            return jnp.logical_and(t >= 0, (wid + t * NW) < NBATCH)

        def boff(t):
            return (wid + t * NW) * EB

        def sidx_copy(t, s):
            return pltpu.make_async_copy(
                src_h.at[pl.ds(boff(t), EB)], sidx.at[s], sem_i.at[s])

        def didx_copy(t, s):
            return pltpu.make_async_copy(
                dst_h.at[pl.ds(boff(t), EB)], didx.at[s], sem_i.at[s])

        def gather_copy(s):
            return pltpu.make_async_copy(
                table_h.at[sidx.at[s]], rows.at[s], sem_g.at[s])

        def sc_desc(s):
            return pltpu.make_async_copy(
                rows.at[s], acc.at[didx.at[s]], sem_sc.at[s])

        def issue_idx(t, s):
            @pl.when(valid(t))
            def _():
                sidx_copy(t, s).start()
                didx_copy(t, s).start()

        def wait_sc(t, s):
            @pl.when(valid(t))
            def _():
                sc_desc(s).wait()

        def start_gather(t, s):
            @pl.when(valid(t))
            def _():
                sidx_copy(t, s).wait()
                didx_copy(t, s).wait()
                gather_copy(s).start()

        def scatter(t, s):
            @pl.when(valid(t))
            def _():
                gather_copy(s).wait()
                pltpu.async_copy(rows.at[s], acc.at[didx.at[s]],
                                 sem_sc.at[s], add=True)

        issue_idx(0, 0)
        issue_idx(1, 1)

        def body(T4, carry):
            T = T4 * 4
            for s in range(4):
                t = T + s
                wait_sc(t - 2, (s + 2) % 4)
                issue_idx(t + 2, (s + 2) % 4)
                start_gather(t, s)
                scatter(t - 1, (s + 3) % 4)
            return carry

        lax.fori_loop(0, BPT // 4 + 1, body, 0)
        wait_sc(BPT - 1, (BPT - 1) % 4)
        plsc.subcore_barrier()
        pltpu.sync_copy(acc.at[pl.ds(sid * RPS, RPS)],
                        out_h.at[pl.ds(cid * NP + sid * RPS, RPS)])

    return k(table, src, dst, zeros_rows)


def _tc_prep1(degp, x, w1):
    def body(degp_ref, x_ref, w_ref, g_ref, dinv_ref):
        deg = degp_ref[0] + degp_ref[1] + 1.0
        dinv = lax.rsqrt(deg)
        h = jnp.dot(x_ref[...], w_ref[...],
                    preferred_element_type=jnp.float32)
        g_ref[...] = h * dinv
        dinv_ref[...] = dinv

    return pl.pallas_call(
        body,
        out_shape=(jax.ShapeDtypeStruct((N, 16), jnp.float32),
                   jax.ShapeDtypeStruct((N, 16), jnp.float32)),
    )(degp, x, w1)


def _tc_mid(accp, g1, dinv, w2p, b1r):
    def body(accp_ref, g_ref, dinv_ref, w_ref, b_ref, g2_ref):
        z = dinv_ref[...] * (accp_ref[0] + accp_ref[1] + g_ref[...])
        z = z + b_ref[...]
        h = jnp.maximum(z, 0.0)
        h2 = jnp.dot(h, w_ref[...], preferred_element_type=jnp.float32)
        g2_ref[...] = h2 * dinv_ref[...]

    return pl.pallas_call(
        body,
        out_shape=jax.ShapeDtypeStruct((N, 16), jnp.float32),
    )(accp, g1, dinv, w2p, b1r)


def _tc_final(accp, g2, dinv, b2r):
    def body(accp_ref, g_ref, dinv_ref, b_ref, o_ref):
        z = dinv_ref[...] * (accp_ref[0] + accp_ref[1] + g_ref[...])
        z = z + b_ref[...]
        col = lax.broadcasted_iota(jnp.int32, (N, 16), 1)
        zm = jnp.where(col < D_OUT, z, -jnp.inf)
        m = jnp.max(zm, axis=1, keepdims=True)
        e = jnp.exp(zm - m)
        lse = jnp.log(jnp.sum(e, axis=1, keepdims=True)) + m
        o_ref[...] = (z - lse)[:, :D_OUT]

    return pl.pallas_call(
        body,
        out_shape=jax.ShapeDtypeStruct((N, D_OUT), jnp.float32),
    )(accp, g2, dinv, b2r)


@jax.jit
def kernel(x, edge_index, W1, b1, W2, b2):
    src = edge_index[0].astype(jnp.int32)
    dst = edge_index[1].astype(jnp.int32)
    zeros_rows = jnp.zeros((RPS, 16), jnp.float32)
    ones_rows = jnp.ones((EB, 16), jnp.float32)

    degp = _deg_scatter(dst, ones_rows, zeros_rows).reshape(NC, NP, 16)[:, :N]
    g1, dinv = _tc_prep1(degp, x, W1)

    acc1 = _edge_scatter(g1, src, dst, zeros_rows).reshape(NC, NP, 16)[:, :N]

    w2p = jnp.pad(W2, ((0, 0), (0, 16 - D_OUT)))
    b1r = b1.reshape(1, 16)
    b2r = jnp.pad(b2, (0, 16 - D_OUT)).reshape(1, 16)

    g2 = _tc_mid(acc1, g1, dinv, w2p, b1r)
    acc2 = _edge_scatter(g2, src, dst, zeros_rows).reshape(NC, NP, 16)[:, :N]
    return _tc_final(acc2, g2, dinv, b2r)


# speedup vs baseline: 48.3946x; 1.1530x over previous
import functools

import jax
import jax.numpy as jnp
from jax import lax
from jax.experimental import pallas as pl
from jax.experimental.pallas import tpu as pltpu
from jax.experimental.pallas import tpu_sc as plsc

N = 10000
E = 320000
D_IN = 128
D_HID = 16
D_OUT = 7

NC = 2
NS = 16
NW = NC * NS
EB = 128
NBATCH = E // EB
BPT = -(-NBATCH // NW)
NP = 10240
RPS = NP // NS


def _sc_mesh():
    return plsc.VectorSubcoreMesh(
        core_axis_name="c", subcore_axis_name="s",
        num_cores=NC, num_subcores=NS)


def _deg_scatter(edge, ones_rows, zeros_rows):
    @functools.partial(
        pl.kernel,
        out_type=jax.ShapeDtypeStruct((NC * NP, 16), jnp.float32),
        mesh=_sc_mesh(),
        scratch_types=[
            pltpu.VMEM((4, EB), jnp.int32),
            pltpu.VMEM((EB, 16), jnp.float32),
            pltpu.VMEM_SHARED((NP, 16), jnp.float32),
            pltpu.SemaphoreType.DMA((4,)),
            pltpu.SemaphoreType.DMA((4,)),
        ],
        compiler_params=pltpu.CompilerParams(use_tc_tiling_on_sc=False),
    )
    def k(edge_h, ones_h, zeros_h, out_h, didx, ones_v, acc, sem_i, sem_sc):
        cid = lax.axis_index("c")
        sid = lax.axis_index("s")
        wid = sid * NC + cid
        pltpu.sync_copy(ones_h, ones_v)
        pltpu.sync_copy(zeros_h, acc.at[pl.ds(sid * RPS, RPS)])
        plsc.subcore_barrier()

        def valid(t):
            return jnp.logical_and(t >= 0, (wid + t * NW) < NBATCH)

        def boff(t):
            return (wid + t * NW) * EB

        def idx_copy(t, s):
            return pltpu.make_async_copy(
                edge_h.at[1, pl.ds(boff(t), EB)], didx.at[s], sem_i.at[s])

        def sc_desc(s):
            return pltpu.make_async_copy(
                ones_v, acc.at[didx.at[s]], sem_sc.at[s])

        def issue_idx(t, s):
            @pl.when(valid(t))
            def _():
                idx_copy(t, s).start()

        def wait_sc(t, s):
            @pl.when(valid(t))
            def _():
                sc_desc(s).wait()

        def scatter(t, s):
            @pl.when(valid(t))
            def _():
                idx_copy(t, s).wait()
                pltpu.async_copy(ones_v, acc.at[didx.at[s]], sem_sc.at[s],
                                 add=True)

        issue_idx(0, 0)
        issue_idx(1, 1)

        def body(T4, carry):
            T = T4 * 4
            for s in range(4):
                t = T + s
                wait_sc(t - 2, (s + 2) % 4)
                issue_idx(t + 2, (s + 2) % 4)
                scatter(t, s)
            return carry

        lax.fori_loop(0, BPT // 4 + 1, body, 0)
        wait_sc(BPT - 1, (BPT - 1) % 4)
        wait_sc(BPT, BPT % 4)
        plsc.subcore_barrier()
        pltpu.sync_copy(acc.at[pl.ds(sid * RPS, RPS)],
                        out_h.at[pl.ds(cid * NP + sid * RPS, RPS)])

    return k(edge, ones_rows, zeros_rows)


def _edge_scatter(table, edge, zeros_rows):
    @functools.partial(
        pl.kernel,
        out_type=jax.ShapeDtypeStruct((NC * NP, 16), jnp.float32),
        mesh=_sc_mesh(),
        scratch_types=[
            pltpu.VMEM((4, EB), jnp.int32),
            pltpu.VMEM((4, EB), jnp.int32),
            pltpu.VMEM((4, EB, 16), jnp.float32),
            pltpu.VMEM_SHARED((NP, 16), jnp.float32),
            pltpu.SemaphoreType.DMA((4,)),
            pltpu.SemaphoreType.DMA((4,)),
            pltpu.SemaphoreType.DMA((4,)),
        ],
        compiler_params=pltpu.CompilerParams(use_tc_tiling_on_sc=False),
    )
    def k(table_h, edge_h, zeros_h, out_h,
          sidx, didx, rows, acc, sem_i, sem_g, sem_sc):
        cid = lax.axis_index("c")
        sid = lax.axis_index("s")
        wid = sid * NC + cid
        pltpu.sync_copy(zeros_h, acc.at[pl.ds(sid * RPS, RPS)])
        plsc.subcore_barrier()

        def valid(t):
            return jnp.logical_and(t >= 0, (wid + t * NW) < NBATCH)

        def boff(t):
            return (wid + t * NW) * EB

        def sidx_copy(t, s):
            return pltpu.make_async_copy(
                edge_h.at[0, pl.ds(boff(t), EB)], sidx.at[s], sem_i.at[s])

        def didx_copy(t, s):
            return pltpu.make_async_copy(
                edge_h.at[1, pl.ds(boff(t), EB)], didx.at[s], sem_i.at[s])

        def gather_copy(s):
            return pltpu.make_async_copy(
                table_h.at[sidx.at[s]], rows.at[s], sem_g.at[s])

        def sc_desc(s):
            return pltpu.make_async_copy(
                rows.at[s], acc.at[didx.at[s]], sem_sc.at[s])

        def issue_idx(t, s):
            @pl.when(valid(t))
            def _():
                sidx_copy(t, s).start()
                didx_copy(t, s).start()

        def wait_sc(t, s):
            @pl.when(valid(t))
            def _():
                sc_desc(s).wait()

        def start_gather(t, s):
            @pl.when(valid(t))
            def _():
                sidx_copy(t, s).wait()
                didx_copy(t, s).wait()
                gather_copy(s).start()

        def scatter(t, s):
            @pl.when(valid(t))
            def _():
                gather_copy(s).wait()
                pltpu.async_copy(rows.at[s], acc.at[didx.at[s]],
                                 sem_sc.at[s], add=True)

        issue_idx(0, 0)
        issue_idx(1, 1)

        def body(T4, carry):
            T = T4 * 4
            for s in range(4):
                t = T + s
                wait_sc(t - 2, (s + 2) % 4)
                issue_idx(t + 2, (s + 2) % 4)
                start_gather(t, s)
                scatter(t - 1, (s + 3) % 4)
            return carry

        lax.fori_loop(0, BPT // 4 + 1, body, 0)
        wait_sc(BPT - 1, (BPT - 1) % 4)
        plsc.subcore_barrier()
        pltpu.sync_copy(acc.at[pl.ds(sid * RPS, RPS)],
                        out_h.at[pl.ds(cid * NP + sid * RPS, RPS)])

    return k(table, edge, zeros_rows)


def _tc_xw1(x, w1):
    def body(x_ref, w_ref, o_ref):
        o_ref[...] = jnp.dot(x_ref[...], w_ref[...],
                             preferred_element_type=jnp.float32)

    return pl.pallas_call(
        body,
        out_shape=jax.ShapeDtypeStruct((N, 16), jnp.float32),
    )(x, w1)


def _tc_scale(degp, xw1):
    def body(degp_ref, xw_ref, g_ref, dinv_ref):
        deg = degp_ref[pl.ds(0, N)] + degp_ref[pl.ds(NP, N)] + 1.0
        dinv = lax.rsqrt(deg)
        g_ref[...] = xw_ref[...] * dinv
        dinv_ref[...] = dinv

    return pl.pallas_call(
        body,
        out_shape=(jax.ShapeDtypeStruct((N, 16), jnp.float32),
                   jax.ShapeDtypeStruct((N, 16), jnp.float32)),
    )(degp, xw1)


def _tc_mid(accp, g1, dinv, w2p, b1r):
    def body(accp_ref, g_ref, dinv_ref, w_ref, b_ref, g2_ref):
        z = dinv_ref[...] * (
            accp_ref[pl.ds(0, N)] + accp_ref[pl.ds(NP, N)] + g_ref[...])
        z = z + b_ref[...]
        h = jnp.maximum(z, 0.0)
        h2 = jnp.dot(h, w_ref[...], preferred_element_type=jnp.float32)
        g2_ref[...] = h2 * dinv_ref[...]

    return pl.pallas_call(
        body,
        out_shape=jax.ShapeDtypeStruct((N, 16), jnp.float32),
    )(accp, g1, dinv, w2p, b1r)


def _tc_final(accp, g2, dinv, b2r):
    def body(accp_ref, g_ref, dinv_ref, b_ref, o_ref):
        z = dinv_ref[...] * (
            accp_ref[pl.ds(0, N)] + accp_ref[pl.ds(NP, N)] + g_ref[...])
        z = z + b_ref[...]
        col = lax.broadcasted_iota(jnp.int32, (N, 16), 1)
        zm = jnp.where(col < D_OUT, z, -jnp.inf)
        m = jnp.max(zm, axis=1, keepdims=True)
        e = jnp.exp(zm - m)
        lse = jnp.log(jnp.sum(e, axis=1, keepdims=True)) + m
        o_ref[...] = (z - lse)[:, :D_OUT]

    return pl.pallas_call(
        body,
        out_shape=jax.ShapeDtypeStruct((N, D_OUT), jnp.float32),
    )(accp, g2, dinv, b2r)


@jax.jit
def kernel(x, edge_index, W1, b1, W2, b2):
    edge = edge_index.astype(jnp.int32)
    zeros_rows = jnp.zeros((RPS, 16), jnp.float32)
    ones_rows = jnp.ones((EB, 16), jnp.float32)

    xw1 = _tc_xw1(x, W1)
    degp = _deg_scatter(edge, ones_rows, zeros_rows)
    g1, dinv = _tc_scale(degp, xw1)

    acc1 = _edge_scatter(g1, edge, zeros_rows)

    w2p = jnp.pad(W2, ((0, 0), (0, 16 - D_OUT)))
    b1r = b1.reshape(1, 16)
    b2r = jnp.pad(b2, (0, 16 - D_OUT)).reshape(1, 16)

    g2 = _tc_mid(acc1, g1, dinv, w2p, b1r)
    acc2 = _edge_scatter(g2, edge, zeros_rows)
    return _tc_final(acc2, g2, dinv, b2r)


# speedup vs baseline: 61.5776x; 1.2724x over previous
import functools

import jax
import jax.numpy as jnp
from jax import lax
from jax.experimental import pallas as pl
from jax.experimental.pallas import tpu as pltpu
from jax.experimental.pallas import tpu_sc as plsc

N = 10000
E = 320000
D_IN = 128
D_HID = 16
D_OUT = 7

NC = 2
NS = 16
NW = NC * NS
EB = 128
NBATCH = E // EB
BPT = -(-NBATCH // NW)
NP = 10240
RPS = NP // NS


def _sc_mesh():
    return plsc.VectorSubcoreMesh(
        core_axis_name="c", subcore_axis_name="s",
        num_cores=NC, num_subcores=NS)


def _deg_scatter(edge, ones_rows, zeros_rows):
    @functools.partial(
        pl.kernel,
        out_type=jax.ShapeDtypeStruct((NC * NP, 16), jnp.float32),
        mesh=_sc_mesh(),
        scratch_types=[
            pltpu.VMEM((4, EB), jnp.int32),
            pltpu.VMEM((EB, 16), jnp.float32),
            pltpu.VMEM_SHARED((NP, 16), jnp.float32),
            pltpu.SemaphoreType.DMA((4,)),
            pltpu.SemaphoreType.DMA((4,)),
        ],
        compiler_params=pltpu.CompilerParams(use_tc_tiling_on_sc=False),
    )
    def k(edge_h, ones_h, zeros_h, out_h, didx, ones_v, acc, sem_i, sem_sc):
        cid = lax.axis_index("c")
        sid = lax.axis_index("s")
        wid = sid * NC + cid
        pltpu.sync_copy(ones_h, ones_v)
        pltpu.sync_copy(zeros_h, acc.at[pl.ds(sid * RPS, RPS)])
        plsc.subcore_barrier()

        def valid(t):
            return jnp.logical_and(t >= 0, (wid + t * NW) < NBATCH)

        def boff(t):
            return (wid + t * NW) * EB

        def idx_copy(t, s):
            return pltpu.make_async_copy(
                edge_h.at[1, pl.ds(boff(t), EB)], didx.at[s], sem_i.at[s])

        def sc_desc(s):
            return pltpu.make_async_copy(
                ones_v, acc.at[didx.at[s]], sem_sc.at[s])

        def issue_idx(t, s):
            @pl.when(valid(t))
            def _():
                idx_copy(t, s).start()

        def wait_sc(t, s):
            @pl.when(valid(t))
            def _():
                sc_desc(s).wait()

        def scatter(t, s):
            @pl.when(valid(t))
            def _():
                idx_copy(t, s).wait()
                pltpu.async_copy(ones_v, acc.at[didx.at[s]], sem_sc.at[s],
                                 add=True)

        issue_idx(0, 0)
        issue_idx(1, 1)

        def body(T4, carry):
            T = T4 * 4
            for s in range(4):
                t = T + s
                wait_sc(t - 2, (s + 2) % 4)
                issue_idx(t + 2, (s + 2) % 4)
                scatter(t, s)
            return carry

        lax.fori_loop(0, BPT // 4 + 1, body, 0)
        wait_sc(BPT - 1, (BPT - 1) % 4)
        wait_sc(BPT, BPT % 4)
        plsc.subcore_barrier()
        pltpu.sync_copy(acc.at[pl.ds(sid * RPS, RPS)],
                        out_h.at[pl.ds(cid * NP + sid * RPS, RPS)])

    return k(edge, ones_rows, zeros_rows)


def _edge_scatter(table, edge, zeros_rows):
    @functools.partial(
        pl.kernel,
        out_type=jax.ShapeDtypeStruct((NC * NP, 16), jnp.float32),
        mesh=_sc_mesh(),
        scratch_types=[
            pltpu.VMEM((4, EB), jnp.int32),
            pltpu.VMEM((4, EB), jnp.int32),
            pltpu.VMEM((4, EB, 16), jnp.float32),
            pltpu.VMEM_SHARED((NP, 16), jnp.float32),
            pltpu.SemaphoreType.DMA((4,)),
            pltpu.SemaphoreType.DMA((4,)),
            pltpu.SemaphoreType.DMA((4,)),
        ],
        compiler_params=pltpu.CompilerParams(use_tc_tiling_on_sc=False),
    )
    def k(table_h, edge_h, zeros_h, out_h,
          sidx, didx, rows, acc, sem_i, sem_g, sem_sc):
        cid = lax.axis_index("c")
        sid = lax.axis_index("s")
        wid = sid * NC + cid
        pltpu.sync_copy(zeros_h, acc.at[pl.ds(sid * RPS, RPS)])
        plsc.subcore_barrier()

        def valid(t):
            return jnp.logical_and(t >= 0, (wid + t * NW) < NBATCH)

        def boff(t):
            return (wid + t * NW) * EB

        def sidx_copy(t, s):
            return pltpu.make_async_copy(
                edge_h.at[0, pl.ds(boff(t), EB)], sidx.at[s], sem_i.at[s])

        def didx_copy(t, s):
            return pltpu.make_async_copy(
                edge_h.at[1, pl.ds(boff(t), EB)], didx.at[s], sem_i.at[s])

        def gather_copy(s):
            return pltpu.make_async_copy(
                table_h.at[sidx.at[s]], rows.at[s], sem_g.at[s])

        def sc_desc(s):
            return pltpu.make_async_copy(
                rows.at[s], acc.at[didx.at[s]], sem_sc.at[s])

        def issue_idx(t, s):
            @pl.when(valid(t))
            def _():
                sidx_copy(t, s).start()
                didx_copy(t, s).start()

        def wait_sc(t, s):
            @pl.when(valid(t))
            def _():
                sc_desc(s).wait()

        def start_gather(t, s):
            @pl.when(valid(t))
            def _():
                sidx_copy(t, s).wait()
                didx_copy(t, s).wait()
                gather_copy(s).start()

        def scatter(t, s):
            @pl.when(valid(t))
            def _():
                gather_copy(s).wait()
                pltpu.async_copy(rows.at[s], acc.at[didx.at[s]],
                                 sem_sc.at[s], add=True)

        issue_idx(0, 0)
        issue_idx(1, 1)

        def body(T4, carry):
            T = T4 * 4
            for s in range(4):
                t = T + s
                wait_sc(t - 2, (s + 2) % 4)
                issue_idx(t + 2, (s + 2) % 4)
                start_gather(t, s)
                scatter(t - 1, (s + 3) % 4)
            return carry

        lax.fori_loop(0, BPT // 4 + 1, body, 0)
        wait_sc(BPT - 1, (BPT - 1) % 4)
        plsc.subcore_barrier()
        pltpu.sync_copy(acc.at[pl.ds(sid * RPS, RPS)],
                        out_h.at[pl.ds(cid * NP + sid * RPS, RPS)])

    return k(table, edge, zeros_rows)


NPK = N // 8
NPPK = NP // 8


def _tc_xw1(x, w1):
    def body(x_ref, w_ref, o_ref):
        o_ref[...] = jnp.dot(x_ref[...], w_ref[...],
                             preferred_element_type=jnp.float32)

    return pl.pallas_call(
        body,
        out_shape=jax.ShapeDtypeStruct((N, 16), jnp.float32),
    )(x, w1)


def _tc_scale(degp, xw1):
    def body(degp_ref, xw_ref, g_ref, dinv_ref):
        deg = degp_ref[pl.ds(0, NPK)] + degp_ref[pl.ds(NPPK, NPK)] + 1.0
        dinv = lax.rsqrt(deg)
        g_ref[...] = xw_ref[...] * dinv
        dinv_ref[...] = dinv

    return pl.pallas_call(
        body,
        out_shape=(jax.ShapeDtypeStruct((NPK, 128), jnp.float32),
                   jax.ShapeDtypeStruct((NPK, 128), jnp.float32)),
    )(degp, xw1)


def _tc_mid(accp, g1, dinv, w2bd, b1p):
    def body(accp_ref, g_ref, dinv_ref, w_ref, b_ref, g2_ref):
        z = dinv_ref[...] * (
            accp_ref[pl.ds(0, NPK)] + accp_ref[pl.ds(NPPK, NPK)] + g_ref[...])
        z = z + b_ref[...]
        h = jnp.maximum(z, 0.0)
        h2 = jnp.dot(h, w_ref[...], preferred_element_type=jnp.float32)
        g2_ref[...] = h2 * dinv_ref[...]

    return pl.pallas_call(
        body,
        out_shape=jax.ShapeDtypeStruct((NPK, 128), jnp.float32),
    )(accp, g1, dinv, w2bd, b1p)


def _tc_zfinal(accp, g2, dinv, b2p):
    def body(accp_ref, g_ref, dinv_ref, b_ref, o_ref):
        zp = dinv_ref[...] * (
            accp_ref[pl.ds(0, NPK)] + accp_ref[pl.ds(NPPK, NPK)] + g_ref[...])
        o_ref[...] = zp + b_ref[...]

    return pl.pallas_call(
        body,
        out_shape=jax.ShapeDtypeStruct((NPK, 128), jnp.float32),
    )(accp, g2, dinv, b2p)


def _tc_softmax(z):
    def body(z_ref, o_ref):
        z = z_ref[...]
        col = lax.broadcasted_iota(jnp.int32, (N, 16), 1)
        zm = jnp.where(col < D_OUT, z, -jnp.inf)
        m = jnp.max(zm, axis=1, keepdims=True)
        e = jnp.exp(zm - m)
        lse = jnp.log(jnp.sum(e, axis=1, keepdims=True)) + m
        o_ref[...] = (z - lse)[:, :D_OUT]

    return pl.pallas_call(
        body,
        out_shape=jax.ShapeDtypeStruct((N, D_OUT), jnp.float32),
    )(z)


@jax.jit
def kernel(x, edge_index, W1, b1, W2, b2):
    edge = edge_index.astype(jnp.int32)
    zeros_rows = jnp.zeros((RPS, 16), jnp.float32)
    ones_rows = jnp.ones((EB, 16), jnp.float32)

    xw1 = _tc_xw1(x, W1).reshape(NPK, 128)
    degp = _deg_scatter(edge, ones_rows, zeros_rows)
    g1, dinv = _tc_scale(degp.reshape(2 * NPPK, 128), xw1)

    acc1 = _edge_scatter(g1.reshape(N, 16), edge, zeros_rows)

    w2p = jnp.pad(W2, ((0, 0), (0, 16 - D_OUT)))
    w2bd = jnp.kron(jnp.eye(8, dtype=jnp.float32), w2p)
    b1p = jnp.tile(b1, 8).reshape(1, 128)
    b2p = jnp.tile(jnp.pad(b2, (0, 16 - D_OUT)), 8).reshape(1, 128)

    g2 = _tc_mid(acc1.reshape(2 * NPPK, 128), g1, dinv, w2bd, b1p)
    acc2 = _edge_scatter(g2.reshape(N, 16), edge, zeros_rows)
    z2 = _tc_zfinal(acc2.reshape(2 * NPPK, 128), g2, dinv, b2p)
    return _tc_softmax(z2.reshape(N, 16))


# speedup vs baseline: 80.1850x; 1.3022x over previous
import functools

import jax
import jax.numpy as jnp
from jax import lax
from jax.experimental import pallas as pl
from jax.experimental.pallas import tpu as pltpu
from jax.experimental.pallas import tpu_sc as plsc

N = 10000
E = 320000
D_IN = 128
D_HID = 16
D_OUT = 7

NC = 2
NS = 16
NW = NC * NS
EB = 128
NBATCH = E // EB
BPT = -(-NBATCH // NW)
NP = 10240
RPS = NP // NS


def _sc_mesh():
    return plsc.VectorSubcoreMesh(
        core_axis_name="c", subcore_axis_name="s",
        num_cores=NC, num_subcores=NS)


def _deg_scatter(edge, ones_rows, zeros_rows):
    @functools.partial(
        pl.kernel,
        out_type=jax.ShapeDtypeStruct((NC * NP, 16), jnp.float32),
        mesh=_sc_mesh(),
        scratch_types=[
            pltpu.VMEM((4, EB), jnp.int32),
            pltpu.VMEM((EB, 16), jnp.float32),
            pltpu.VMEM_SHARED((NP, 16), jnp.float32),
            pltpu.SemaphoreType.DMA((4,)),
            pltpu.SemaphoreType.DMA((4,)),
        ],
        compiler_params=pltpu.CompilerParams(use_tc_tiling_on_sc=False),
    )
    def k(edge_h, ones_h, zeros_h, out_h, didx, ones_v, acc, sem_i, sem_sc):
        cid = lax.axis_index("c")
        sid = lax.axis_index("s")
        wid = sid * NC + cid
        pltpu.sync_copy(ones_h, ones_v)
        pltpu.sync_copy(zeros_h, acc.at[pl.ds(sid * RPS, RPS)])
        plsc.subcore_barrier()

        def valid(t):
            return jnp.logical_and(t >= 0, (wid + t * NW) < NBATCH)

        def boff(t):
            return (wid + t * NW) * EB

        def idx_copy(t, s):
            return pltpu.make_async_copy(
                edge_h.at[1, pl.ds(boff(t), EB)], didx.at[s], sem_i.at[s])

        def sc_desc(s):
            return pltpu.make_async_copy(
                ones_v, acc.at[didx.at[s]], sem_sc.at[s])

        def issue_idx(t, s):
            @pl.when(valid(t))
            def _():
                idx_copy(t, s).start()

        def wait_sc(t, s):
            @pl.when(valid(t))
            def _():
                sc_desc(s).wait()

        def scatter(t, s):
            @pl.when(valid(t))
            def _():
                idx_copy(t, s).wait()
                pltpu.async_copy(ones_v, acc.at[didx.at[s]], sem_sc.at[s],
                                 add=True)

        issue_idx(0, 0)
        issue_idx(1, 1)

        def body(T4, carry):
            T = T4 * 4
            for s in range(4):
                t = T + s
                wait_sc(t - 2, (s + 2) % 4)
                issue_idx(t + 2, (s + 2) % 4)
                scatter(t, s)
            return carry

        lax.fori_loop(0, BPT // 4 + 1, body, 0)
        wait_sc(BPT - 1, (BPT - 1) % 4)
        wait_sc(BPT, BPT % 4)
        plsc.subcore_barrier()
        pltpu.sync_copy(acc.at[pl.ds(sid * RPS, RPS)],
                        out_h.at[pl.ds(cid * NP + sid * RPS, RPS)])

    return k(edge, ones_rows, zeros_rows)


def _edge_scatter(table, edge, zeros_rows):
    @functools.partial(
        pl.kernel,
        out_type=jax.ShapeDtypeStruct((NC * NP, 16), jnp.float32),
        mesh=_sc_mesh(),
        scratch_types=[
            pltpu.VMEM((4, EB), jnp.int32),
            pltpu.VMEM((4, EB), jnp.int32),
            pltpu.VMEM((4, EB, 16), jnp.float32),
            pltpu.VMEM_SHARED((NP, 16), jnp.float32),
            pltpu.VMEM_SHARED((N, 16), jnp.float32),
            pltpu.SemaphoreType.DMA((4,)),
            pltpu.SemaphoreType.DMA((4,)),
            pltpu.SemaphoreType.DMA((4,)),
        ],
        compiler_params=pltpu.CompilerParams(use_tc_tiling_on_sc=False),
    )
    def k(table_h, edge_h, zeros_h, out_h,
          sidx, didx, rows, acc, tbl, sem_i, sem_g, sem_sc):
        cid = lax.axis_index("c")
        sid = lax.axis_index("s")
        wid = sid * NC + cid
        pltpu.sync_copy(zeros_h, acc.at[pl.ds(sid * RPS, RPS)])
        @pl.when(sid < 15)
        def _():
            pltpu.sync_copy(table_h.at[pl.ds(sid * 624, 624)],
                            tbl.at[pl.ds(sid * 624, 624)])

        @pl.when(sid == 15)
        def _():
            pltpu.sync_copy(table_h.at[pl.ds(9360, 640)],
                            tbl.at[pl.ds(9360, 640)])

        plsc.subcore_barrier()

        def valid(t):
            return jnp.logical_and(t >= 0, (wid + t * NW) < NBATCH)

        def boff(t):
            return (wid + t * NW) * EB

        def sidx_copy(t, s):
            return pltpu.make_async_copy(
                edge_h.at[0, pl.ds(boff(t), EB)], sidx.at[s], sem_i.at[s])

        def didx_copy(t, s):
            return pltpu.make_async_copy(
                edge_h.at[1, pl.ds(boff(t), EB)], didx.at[s], sem_i.at[s])

        def gather_copy(s):
            return pltpu.make_async_copy(
                tbl.at[sidx.at[s]], rows.at[s], sem_g.at[s])

        def sc_desc(s):
            return pltpu.make_async_copy(
                rows.at[s], acc.at[didx.at[s]], sem_sc.at[s])

        def issue_idx(t, s):
            @pl.when(valid(t))
            def _():
                sidx_copy(t, s).start()
                didx_copy(t, s).start()

        def wait_sc(t, s):
            @pl.when(valid(t))
            def _():
                sc_desc(s).wait()

        def start_gather(t, s):
            @pl.when(valid(t))
            def _():
                sidx_copy(t, s).wait()
                didx_copy(t, s).wait()
                gather_copy(s).start()

        def scatter(t, s):
            @pl.when(valid(t))
            def _():
                gather_copy(s).wait()
                pltpu.async_copy(rows.at[s], acc.at[didx.at[s]],
                                 sem_sc.at[s], add=True)

        issue_idx(0, 0)
        issue_idx(1, 1)

        def body(T4, carry):
            T = T4 * 4
            for s in range(4):
                t = T + s
                wait_sc(t - 2, (s + 2) % 4)
                issue_idx(t + 2, (s + 2) % 4)
                start_gather(t, s)
                scatter(t - 1, (s + 3) % 4)
            return carry

        lax.fori_loop(0, BPT // 4 + 1, body, 0)
        wait_sc(BPT - 1, (BPT - 1) % 4)
        plsc.subcore_barrier()
        pltpu.sync_copy(acc.at[pl.ds(sid * RPS, RPS)],
                        out_h.at[pl.ds(cid * NP + sid * RPS, RPS)])

    return k(table, edge, zeros_rows)


NPK = N // 8
NPPK = NP // 8


def _tc_xw1(x, w1):
    def body(x_ref, w_ref, o_ref):
        o_ref[...] = jnp.dot(x_ref[...], w_ref[...],
                             preferred_element_type=jnp.float32)

    return pl.pallas_call(
        body,
        out_shape=jax.ShapeDtypeStruct((N, 16), jnp.float32),
    )(x, w1)


def _tc_scale(degp, xw1):
    def body(degp_ref, xw_ref, g_ref, dinv_ref):
        deg = degp_ref[pl.ds(0, NPK)] + degp_ref[pl.ds(NPPK, NPK)] + 1.0
        dinv = lax.rsqrt(deg)
        g_ref[...] = xw_ref[...] * dinv
        dinv_ref[...] = dinv

    return pl.pallas_call(
        body,
        out_shape=(jax.ShapeDtypeStruct((NPK, 128), jnp.float32),
                   jax.ShapeDtypeStruct((NPK, 128), jnp.float32)),
    )(degp, xw1)


def _tc_mid(accp, g1, dinv, w2bd, b1p):
    def body(accp_ref, g_ref, dinv_ref, w_ref, b_ref, g2_ref):
        z = dinv_ref[...] * (
            accp_ref[pl.ds(0, NPK)] + accp_ref[pl.ds(NPPK, NPK)] + g_ref[...])
        z = z + b_ref[...]
        h = jnp.maximum(z, 0.0)
        h2 = jnp.dot(h, w_ref[...], preferred_element_type=jnp.float32)
        g2_ref[...] = h2 * dinv_ref[...]

    return pl.pallas_call(
        body,
        out_shape=jax.ShapeDtypeStruct((NPK, 128), jnp.float32),
    )(accp, g1, dinv, w2bd, b1p)


def _tc_zfinal(accp, g2, dinv, b2p):
    def body(accp_ref, g_ref, dinv_ref, b_ref, o_ref):
        zp = dinv_ref[...] * (
            accp_ref[pl.ds(0, NPK)] + accp_ref[pl.ds(NPPK, NPK)] + g_ref[...])
        o_ref[...] = zp + b_ref[...]

    return pl.pallas_call(
        body,
        out_shape=jax.ShapeDtypeStruct((NPK, 128), jnp.float32),
    )(accp, g2, dinv, b2p)


def _tc_softmax(z):
    def body(z_ref, o_ref):
        z = z_ref[...]
        col = lax.broadcasted_iota(jnp.int32, (N, 16), 1)
        zm = jnp.where(col < D_OUT, z, -jnp.inf)
        m = jnp.max(zm, axis=1, keepdims=True)
        e = jnp.exp(zm - m)
        lse = jnp.log(jnp.sum(e, axis=1, keepdims=True)) + m
        o_ref[...] = (z - lse)[:, :D_OUT]

    return pl.pallas_call(
        body,
        out_shape=jax.ShapeDtypeStruct((N, D_OUT), jnp.float32),
    )(z)


@jax.jit
def kernel(x, edge_index, W1, b1, W2, b2):
    edge = edge_index.astype(jnp.int32)
    zeros_rows = jnp.zeros((RPS, 16), jnp.float32)
    ones_rows = jnp.ones((EB, 16), jnp.float32)

    xw1 = _tc_xw1(x, W1).reshape(NPK, 128)
    degp = _deg_scatter(edge, ones_rows, zeros_rows)
    g1, dinv = _tc_scale(degp.reshape(2 * NPPK, 128), xw1)

    acc1 = _edge_scatter(g1.reshape(N, 16), edge, zeros_rows)

    w2p = jnp.pad(W2, ((0, 0), (0, 16 - D_OUT)))
    w2bd = jnp.kron(jnp.eye(8, dtype=jnp.float32), w2p)
    b1p = jnp.tile(b1, 8).reshape(1, 128)
    b2p = jnp.tile(jnp.pad(b2, (0, 16 - D_OUT)), 8).reshape(1, 128)

    g2 = _tc_mid(acc1.reshape(2 * NPPK, 128), g1, dinv, w2bd, b1p)
    acc2 = _edge_scatter(g2.reshape(N, 16), edge, zeros_rows)
    z2 = _tc_zfinal(acc2.reshape(2 * NPPK, 128), g2, dinv, b2p)
    return _tc_softmax(z2.reshape(N, 16))
